# baseline (speedup 1.0000x reference)
import functools

import jax
import jax.numpy as jnp
from jax import lax
from jax.experimental import pallas as pl
from jax.experimental.pallas import tpu as pltpu
from jax.experimental.pallas import tpu_sc as plsc

_NC = 2
_NS = 16
_NW = _NC * _NS
_L = 16

_A = 4096
_P = 26
_S = 20
_C = _P * _S
_B = _A * _C
_D = 64
_AB = _A // _NW
_NG = _AB // _L

_ZROW = 42
_VPAD = 48


@functools.partial(
    pl.kernel,
    out_type=(
        jax.ShapeDtypeStruct((_P, _S, _D, _A), jnp.float32),
        jax.ShapeDtypeStruct((_P, _S, _A), jnp.int32),
    ),
    mesh=plsc.VectorSubcoreMesh(core_axis_name="c", subcore_axis_name="s"),
    compiler_params=pltpu.CompilerParams(use_tc_tiling_on_sc=True,
                                         needs_layout_passes=False),
    scratch_types=[
        pltpu.VMEM((_D * _VPAD,), jnp.float32),
        pltpu.VMEM((_AB * _C,), jnp.int32),
        pltpu.VMEM((2, _D, _AB), jnp.float32),
        pltpu.VMEM((2, _AB), jnp.int32),
        pltpu.SemaphoreType.DMA,
        pltpu.SemaphoreType.DMA,
        pltpu.SemaphoreType.DMA,
        pltpu.SemaphoreType.DMA,
    ],
)
def _emb_lookup(tablet_hbm, labels_hbm, out_hbm, mask_hbm,
                tablet_v, labels_v, tile_v, maskv,
                osem0, osem1, msem0, msem1):
    cid = lax.axis_index("c")
    sid = lax.axis_index("s")
    wid = sid * _NC + cid
    a0 = wid * _AB

    pltpu.sync_copy(tablet_hbm, tablet_v)
    pltpu.sync_copy(labels_hbm.at[pl.ds(a0 * _C, _AB * _C)], labels_v)

    osems = (osem0, osem1)
    msems = (msem0, msem1)
    iota_stride = lax.iota(jnp.int32, _L) * _C

    def fire_tile(p, s, eb):
        pltpu.async_copy(tile_v.at[eb],
                         out_hbm.at[p, s, pl.ds(0, _D), pl.ds(a0, _AB)],
                         osems[eb])

    def wait_tile(p, s, eb):
        pltpu.make_async_copy(tile_v.at[eb],
                              out_hbm.at[p, s, pl.ds(0, _D), pl.ds(a0, _AB)],
                              osems[eb]).wait()

    def item(p, s, eb):
        t = p * _S + s

        @pl.when(t >= 2)
        def _():
            wait_tile(p, s, eb)
            wait_mask(p, s, eb)

        for g in range(_NG):
            base = (g * _L) * _C + p * _S + s
            lab = plsc.load_gather(
                labels_v, [iota_stride + jnp.full((_L,), base, jnp.int32)])
            is_pad = lab == jnp.full((_L,), -1, jnp.int32)
            msk = jnp.where(is_pad, jnp.zeros((_L,), jnp.int32),
                            jnp.ones((_L,), jnp.int32))
            fixed = jnp.where(is_pad, jnp.full((_L,), _ZROW, jnp.int32), lab)
            maskv[eb, pl.ds(g * _L, _L)] = msk

            for d in range(_D):
                emb = plsc.load_gather(
                    tablet_v, [fixed + jnp.full((_L,), d * _VPAD, jnp.int32)])
                tile_v[eb, d, pl.ds(g * _L, _L)] = emb

        fire_tile(p, s, eb)
        fire_mask(p, s, eb)

    def fire_mask(p, s, eb):
        pltpu.async_copy(maskv.at[eb],
                         mask_hbm.at[p, s, pl.ds(a0, _AB)], msems[eb])

    def wait_mask(p, s, eb):
        pltpu.make_async_copy(maskv.at[eb],
                              mask_hbm.at[p, s, pl.ds(a0, _AB)],
                              msems[eb]).wait()

    def plane(p):
        def s_pair(u, carry):
            item(p, 2 * u, 0)
            item(p, 2 * u + 1, 1)
            return carry

        lax.fori_loop(0, _S // 2, s_pair, 0)

    def p_body(p, carry):
        plane(p)
        return carry

    lax.fori_loop(0, _P, p_body, 0)
    wait_tile(_P - 1, _S - 2, 0)
    wait_tile(_P - 1, _S - 1, 1)
    wait_mask(_P - 1, _S - 2, 0)
    wait_mask(_P - 1, _S - 1, 1)


def kernel(output, action_emb):
    labels = output[0].reshape(_B)
    table = jnp.concatenate(
        [action_emb, jnp.zeros((_VPAD - action_emb.shape[0], _D), jnp.float32)])
    tablet = jnp.transpose(table).reshape(_D * _VPAD)
    emb_t, mask_t = _emb_lookup(tablet, labels)
    return (jnp.transpose(emb_t, (3, 0, 1, 2)), jnp.transpose(mask_t, (2, 0, 1)))


# speedup vs baseline: 5.5352x; 5.5352x over previous
import functools

import jax
import jax.numpy as jnp
from jax import lax
from jax.experimental import pallas as pl
from jax.experimental.pallas import tpu as pltpu
from jax.experimental.pallas import tpu_sc as plsc

_NC = 2
_NS = 16
_NW = _NC * _NS
_L = 16

_A = 4096
_P = 26
_S = 20
_C = _P * _S
_B = _A * _C
_D = 64
_AB = _A // _NW
_NG = _AB // _L

_ZROW = 42
_VPAD = 48


@functools.partial(
    pl.kernel,
    out_type=(
        jax.ShapeDtypeStruct((_P, _S, _D, _A), jnp.float32),
        jax.ShapeDtypeStruct((_P, _S, _A), jnp.int32),
    ),
    mesh=plsc.VectorSubcoreMesh(core_axis_name="c", subcore_axis_name="s"),
    compiler_params=pltpu.CompilerParams(use_tc_tiling_on_sc=True,
                                         needs_layout_passes=False),
    scratch_types=[
        pltpu.VMEM((_D * _VPAD,), jnp.float32),
        pltpu.VMEM((_AB * _C,), jnp.int32),
        pltpu.VMEM((2, _D, _AB), jnp.float32),
        pltpu.VMEM((2, _AB), jnp.int32),
        pltpu.SemaphoreType.DMA,
        pltpu.SemaphoreType.DMA,
        pltpu.SemaphoreType.DMA,
        pltpu.SemaphoreType.DMA,
    ],
)
def _emb_lookup(tablet_hbm, labels_hbm, out_hbm, mask_hbm,
                tablet_v, labels_v, tile_v, maskv,
                osem0, osem1, msem0, msem1):
    cid = lax.axis_index("c")
    sid = lax.axis_index("s")
    wid = sid * _NC + cid
    a0 = wid * _AB

    pltpu.sync_copy(tablet_hbm, tablet_v)
    pltpu.sync_copy(labels_hbm.at[pl.ds(a0 * _C, _AB * _C)], labels_v)

    osems = (osem0, osem1)
    msems = (msem0, msem1)
    iota_stride = lax.iota(jnp.int32, _L) * _C

    def fire_tile(p, s, eb):
        pltpu.async_copy(tile_v.at[eb],
                         out_hbm.at[p, s, pl.ds(0, _D), pl.ds(a0, _AB)],
                         osems[eb])

    def wait_tile(p, s, eb):
        pltpu.make_async_copy(tile_v.at[eb],
                              out_hbm.at[p, s, pl.ds(0, _D), pl.ds(a0, _AB)],
                              osems[eb]).wait()

    def item(p, s, eb):
        t = p * _S + s

        @pl.when(t >= 2)
        def _():
            wait_tile(p, s, eb)
            wait_mask(p, s, eb)

        fixed_groups = []
        for g in range(_NG):
            base = (g * _L) * _C + p * _S + s
            lab = plsc.load_gather(
                labels_v, [iota_stride + jnp.full((_L,), base, jnp.int32)])
            is_pad = lab == jnp.full((_L,), -1, jnp.int32)
            msk = jnp.where(is_pad, jnp.zeros((_L,), jnp.int32),
                            jnp.ones((_L,), jnp.int32))
            fixed_groups.append(
                jnp.where(is_pad, jnp.full((_L,), _ZROW, jnp.int32), lab))
            maskv[eb, pl.ds(g * _L, _L)] = msk

        @plsc.parallel_loop(0, _D, unroll=4)
        def _(d):
            col = d * _VPAD
            for g in range(_NG):
                emb = plsc.load_gather(
                    tablet_v,
                    [fixed_groups[g] + jnp.full((_L,), col, jnp.int32)])
                tile_v[eb, d, pl.ds(g * _L, _L)] = emb

        fire_tile(p, s, eb)
        fire_mask(p, s, eb)

    def fire_mask(p, s, eb):
        pltpu.async_copy(maskv.at[eb],
                         mask_hbm.at[p, s, pl.ds(a0, _AB)], msems[eb])

    def wait_mask(p, s, eb):
        pltpu.make_async_copy(maskv.at[eb],
                              mask_hbm.at[p, s, pl.ds(a0, _AB)],
                              msems[eb]).wait()

    def plane(p):
        def s_pair(u, carry):
            item(p, 2 * u, 0)
            item(p, 2 * u + 1, 1)
            return carry

        lax.fori_loop(0, _S // 2, s_pair, 0)

    def p_body(p, carry):
        plane(p)
        return carry

    lax.fori_loop(0, _P, p_body, 0)
    wait_tile(_P - 1, _S - 2, 0)
    wait_tile(_P - 1, _S - 1, 1)
    wait_mask(_P - 1, _S - 2, 0)
    wait_mask(_P - 1, _S - 1, 1)


def kernel(output, action_emb):
    labels = output[0].reshape(_B)
    table = jnp.concatenate(
        [action_emb, jnp.zeros((_VPAD - action_emb.shape[0], _D), jnp.float32)])
    tablet = jnp.transpose(table).reshape(_D * _VPAD)
    emb_t, mask_t = _emb_lookup(tablet, labels)
    return (jnp.transpose(emb_t, (3, 0, 1, 2)), jnp.transpose(mask_t, (2, 0, 1)))
